# baseline (speedup 1.0000x reference)
import functools

import jax
import jax.numpy as jnp
from jax import lax
from jax.experimental import pallas as pl
from jax.experimental.pallas import tpu as pltpu
from jax.experimental.pallas import tpu_sc as plsc

_LEVELS = 1024
_OUT = 2048
_N = 224 * 224
_NSUB = 16
_PER_S = _N // _NSUB
_VPS = _PER_S // 16
_NLANES = 16
_TAB = _NLANES * _LEVELS
_RNE = 8388608.0

_mesh = plsc.VectorSubcoreMesh(core_axis_name="c", subcore_axis_name="s")


@functools.partial(
    pl.kernel,
    mesh=_mesh,
    out_type=(
        jax.ShapeDtypeStruct((_OUT,), jnp.float32),
        jax.ShapeDtypeStruct((2, 16, _LEVELS), jnp.float32),
        jax.ShapeDtypeStruct((2, 16, 128), jnp.float32),
    ),
    compiler_params=pltpu.CompilerParams(needs_layout_passes=False),
    scratch_types=[
        pltpu.VMEM((_PER_S,), jnp.float32),
        pltpu.VMEM((_TAB,), jnp.float32),
        pltpu.VMEM((_LEVELS,), jnp.float32),
        pltpu.VMEM((512, 128), jnp.float32),
        pltpu.VMEM((16, _LEVELS), jnp.float32),
        pltpu.VMEM((128,), jnp.float32),
        pltpu.VMEM((64,), jnp.float32),
        pltpu.VMEM((64,), jnp.float32),
        pltpu.SemaphoreType.DMA,
        pltpu.SemaphoreType.DMA,
    ],
)
def _sc_fused(x_hbm, w_hbm, out_hbm, cex_hbm, pex_hbm, x_v, tab_v, cnt_v,
              slab_v, big_v, pout_v, fa_v, fb_v, sem_x, sem_w):
    c = lax.axis_index("c")
    s = lax.axis_index("s")
    g = s // 2
    h = s % 2

    cp_w = pltpu.async_copy(
        w_hbm.at[pl.ds(h * 512, 512), pl.ds(c * 1024 + g * 128, 128)],
        slab_v, sem_w)
    cp_x = pltpu.async_copy(x_hbm.at[pl.ds(s * _PER_S, _PER_S)], x_v, sem_x)

    zeros16 = jnp.zeros((16,), jnp.float32)
    iota16 = lax.iota(jnp.int32, 16)

    def _zero(j, k):
        for t in range(_NLANES):
            tab_v[pl.ds(t * _LEVELS + j * 16, 16)] = zeros16
        return k

    lax.fori_loop(0, _LEVELS // 16, _zero, 0)
    cp_x.wait()

    lane_base = iota16 * _LEVELS
    ones16 = jnp.ones((16,), jnp.float32)

    def _hist(i, k):
        for u in range(2):
            xv = x_v[pl.ds((2 * i + u) * 16, 16)]
            v = xv * float(_LEVELS - 1)
            r = (v + _RNE) - _RNE
            idx = jnp.clip(r.astype(jnp.int32), 0, _LEVELS - 1)
            plsc.addupdate_scatter(tab_v, [lane_base + idx], ones16)
        return k

    lax.fori_loop(0, _VPS // 2, _hist, 0)

    def _red(j, k):
        acc = tab_v[pl.ds(j * 16, 16)]
        for t in range(1, _NLANES):
            acc = acc + tab_v[pl.ds(t * _LEVELS + j * 16, 16)]
        cnt_v[pl.ds(j * 16, 16)] = acc
        return k

    lax.fori_loop(0, _LEVELS // 16, _red, 0)

    pltpu.sync_copy(cnt_v, cex_hbm.at[c, s])
    plsc.subcore_barrier()
    pltpu.sync_copy(cex_hbm.at[c], big_v)

    def _red2(j, k):
        acc = big_v[0, pl.ds(j * 16, 16)]
        for t in range(1, _NLANES):
            acc = acc + big_v[t, pl.ds(j * 16, 16)]
        cnt_v[pl.ds(j * 16, 16)] = acc
        return k

    lax.fori_loop(0, _LEVELS // 16, _red2, 0)

    cp_w.wait()
    zeros16i = jnp.zeros((16,), jnp.int32)

    def _mv(j2, accs):
        row0 = j2 * 16
        lbase = zeros16i + (h * 512 + j2 * 16)
        new = list(accs)
        for t in range(16):
            cb = plsc.load_gather(cnt_v, [lbase + t])
            for k in range(8):
                new[k] = new[k] + cb * slab_v[row0 + t, pl.ds(k * 16, 16)]
        return tuple(new)

    accs = lax.fori_loop(
        0, 32, _mv, tuple(jnp.zeros((16,), jnp.float32) for _ in range(8)))
    for k in range(8):
        pout_v[pl.ds(k * 16, 16)] = accs[k]
    pltpu.sync_copy(pout_v, pex_hbm.at[c, s])
    plsc.subcore_barrier()

    off = (s % 2) * 64
    pltpu.sync_copy(pex_hbm.at[c, 2 * (s // 2), pl.ds(off, 64)], fa_v)
    pltpu.sync_copy(pex_hbm.at[c, 2 * (s // 2) + 1, pl.ds(off, 64)], fb_v)
    for r in range(4):
        fa_v[pl.ds(r * 16, 16)] = (
            fa_v[pl.ds(r * 16, 16)] + fb_v[pl.ds(r * 16, 16)])
    pltpu.sync_copy(fa_v, out_hbm.at[pl.ds(c * 1024 + s * 64, 64)])


def kernel(x, level_weight):
    out, _, _ = _sc_fused(x, level_weight)
    return out


# speedup vs baseline: 1.2355x; 1.2355x over previous
import functools

import jax
import jax.numpy as jnp
from jax import lax
from jax.experimental import pallas as pl
from jax.experimental.pallas import tpu as pltpu
from jax.experimental.pallas import tpu_sc as plsc

_LEVELS = 1024
_OUT = 2048
_N = 224 * 224
_NW = 16
_PER_W = _N // _NW
_VPW = _PER_W // 16
_NLANES = 16
_TAB = _NLANES * _LEVELS
_RNE = 8388608.0

_mesh = plsc.VectorSubcoreMesh(
    core_axis_name="c", subcore_axis_name="s", num_cores=1)


@functools.partial(
    pl.kernel,
    mesh=_mesh,
    out_type=jax.ShapeDtypeStruct((_NW, _LEVELS), jnp.float32),
    compiler_params=pltpu.CompilerParams(needs_layout_passes=False),
    scratch_types=[
        pltpu.VMEM((_PER_W,), jnp.float32),
        pltpu.VMEM((_TAB,), jnp.float32),
        pltpu.VMEM((_LEVELS,), jnp.float32),
        pltpu.SemaphoreType.DMA,
    ],
)
def _sc_hist(x_hbm, out_hbm, x_v, tab_v, cnt_v, sem):
    wid = lax.axis_index("s")
    base = wid * _PER_W
    cp = pltpu.async_copy(x_hbm.at[pl.ds(base, _PER_W)], x_v, sem)

    zeros16 = jnp.zeros((16,), jnp.float32)

    def _zero(j, c):
        for t in range(_NLANES):
            tab_v[pl.ds(t * _LEVELS + j * 16, 16)] = zeros16
        return c

    lax.fori_loop(0, _LEVELS // 16, _zero, 0)
    cp.wait()

    lane_base = lax.iota(jnp.int32, 16) * _LEVELS
    ones16 = jnp.ones((16,), jnp.float32)

    def _hist(i, c):
        for u in range(2):
            xv = x_v[pl.ds((2 * i + u) * 16, 16)]
            v = xv * float(_LEVELS - 1)
            r = (v + _RNE) - _RNE
            idx = jnp.clip(r.astype(jnp.int32), 0, _LEVELS - 1)
            plsc.addupdate_scatter(tab_v, [lane_base + idx], ones16)
        return c

    lax.fori_loop(0, _VPW // 2, _hist, 0)

    def _red(j, c):
        acc = tab_v[pl.ds(j * 16, 16)]
        for t in range(1, _NLANES):
            acc = acc + tab_v[pl.ds(t * _LEVELS + j * 16, 16)]
        cnt_v[pl.ds(j * 16, 16)] = acc
        return c

    lax.fori_loop(0, _LEVELS // 16, _red, 0)

    pltpu.sync_copy(cnt_v, out_hbm.at[wid])


def _mv_body(cp_ref, w_ref, o_ref):
    counts = jnp.sum(cp_ref[...], axis=0, keepdims=True)
    o_ref[...] = jnp.dot(counts, w_ref[...], preferred_element_type=jnp.float32)


def kernel(x, level_weight):
    counts_parts = _sc_hist(x)
    out = pl.pallas_call(
        _mv_body,
        out_shape=jax.ShapeDtypeStruct((1, _OUT), jnp.float32),
    )(counts_parts, level_weight)
    return out.reshape(_OUT)


# speedup vs baseline: 1.3179x; 1.0668x over previous
import functools

import jax
import jax.numpy as jnp
from jax import lax
from jax.experimental import pallas as pl
from jax.experimental.pallas import tpu as pltpu
from jax.experimental.pallas import tpu_sc as plsc

_LEVELS = 1024
_OUT = 2048
_N = 224 * 224
_NW = 16
_PER_W = _N // _NW
_VPW = _PER_W // 16
_NLANES = 16
_TAB = _NLANES * _LEVELS
_RNE = 8388608.0

_mesh = plsc.VectorSubcoreMesh(
    core_axis_name="c", subcore_axis_name="s", num_cores=1)


@functools.partial(
    pl.kernel,
    mesh=_mesh,
    out_type=jax.ShapeDtypeStruct((_NW, _LEVELS), jnp.float32),
    compiler_params=pltpu.CompilerParams(needs_layout_passes=False),
    scratch_types=[
        pltpu.VMEM((_PER_W,), jnp.float32),
        pltpu.VMEM((_LEVELS,), jnp.float32),
        pltpu.SemaphoreType.DMA,
    ],
)
def _sc_hist(x_hbm, out_hbm, x_v, tab_v, sem):
    wid = lax.axis_index("s")
    base = wid * _PER_W
    cp = pltpu.async_copy(x_hbm.at[pl.ds(base, _PER_W)], x_v, sem)

    zeros16 = jnp.zeros((16,), jnp.float32)

    def _zero(j, c):
        tab_v[pl.ds(j * 16, 16)] = zeros16
        return c

    lax.fori_loop(0, _LEVELS // 16, _zero, 0)
    cp.wait()

    ones16 = jnp.ones((16,), jnp.float32)

    def _hist(i, c):
        for u in range(2):
            xv = x_v[pl.ds((2 * i + u) * 16, 16)]
            v = xv * float(_LEVELS - 1)
            r = (v + _RNE) - _RNE
            idx = jnp.clip(r.astype(jnp.int32), 0, _LEVELS - 1)
            plsc.addupdate_scatter(tab_v, [idx], ones16)
        return c

    lax.fori_loop(0, _VPW // 2, _hist, 0)

    pltpu.sync_copy(tab_v, out_hbm.at[wid])


def _mv_body(cp_ref, w_ref, o_ref):
    counts = jnp.sum(cp_ref[...], axis=0, keepdims=True)
    o_ref[...] = jnp.dot(counts, w_ref[...], preferred_element_type=jnp.float32)


def kernel(x, level_weight):
    counts_parts = _sc_hist(x)
    out = pl.pallas_call(
        _mv_body,
        out_shape=jax.ShapeDtypeStruct((1, _OUT), jnp.float32),
    )(counts_parts, level_weight)
    return out.reshape(_OUT)


# speedup vs baseline: 1.3234x; 1.0041x over previous
import functools

import jax
import jax.numpy as jnp
from jax import lax
from jax.experimental import pallas as pl
from jax.experimental.pallas import tpu as pltpu
from jax.experimental.pallas import tpu_sc as plsc

_LEVELS = 1024
_OUT = 2048
_N = 224 * 224
_NW = 16
_PER_W = _N // _NW
_VPW = _PER_W // 16
_RNE = 8388608.0

_mesh = plsc.VectorSubcoreMesh(
    core_axis_name="c", subcore_axis_name="s", num_cores=1)


@functools.partial(
    pl.kernel,
    mesh=_mesh,
    out_type=jax.ShapeDtypeStruct((_NW, _LEVELS), jnp.float32),
    compiler_params=pltpu.CompilerParams(needs_layout_passes=False),
    scratch_types=[
        pltpu.VMEM((_PER_W,), jnp.float32),
        pltpu.VMEM((_LEVELS,), jnp.float32),
        pltpu.SemaphoreType.DMA,
    ],
)
def _sc_hist(x_hbm, out_hbm, x_v, tab_v, sem):
    wid = lax.axis_index("s")
    base = wid * _PER_W
    cp = pltpu.async_copy(x_hbm.at[pl.ds(base, _PER_W)], x_v, sem)

    zeros16 = jnp.zeros((16,), jnp.float32)

    def _zero(j, c):
        tab_v[pl.ds(j * 16, 16)] = zeros16
        return c

    lax.fori_loop(0, _LEVELS // 16, _zero, 0)
    cp.wait()

    ones16 = jnp.ones((16,), jnp.float32)

    def _hist(i, c):
        for u in range(2):
            xv = x_v[pl.ds((2 * i + u) * 16, 16)]
            v = xv * float(_LEVELS - 1)
            r = (v + _RNE) - _RNE
            idx = jnp.clip(r.astype(jnp.int32), 0, _LEVELS - 1)
            plsc.addupdate_scatter(tab_v, [idx], ones16)
        return c

    lax.fori_loop(0, _VPW // 2, _hist, 0)

    pltpu.sync_copy(tab_v, out_hbm.at[wid])


def _mv_body(cp_ref, w_ref, o_ref):
    counts = jnp.sum(cp_ref[...], axis=0, keepdims=True)
    o_ref[...] = jnp.dot(counts, w_ref[...], preferred_element_type=jnp.float32)


def kernel(x, level_weight):
    counts_parts = _sc_hist(x)
    out = pl.pallas_call(
        _mv_body,
        out_shape=jax.ShapeDtypeStruct((1, _OUT), jnp.float32),
    )(counts_parts, level_weight)
    return out.reshape(_OUT)
